# baseline (speedup 1.0000x reference)
import jax
import jax.numpy as jnp
from jax import lax
from jax.experimental import pallas as pl
from jax.experimental.pallas import tpu as pltpu
from jax.experimental.pallas import tpu_sc as plsc

B = 1024
V = 100000
L = 16
NC, NS = 1, 16
NW = NC * NS
BPW = B // NW


def _body(flat_hbm, labels_hbm, tgt_hbm, out_hbm,
          labels_v, tgt_v, idx_v, gt_v, tg_v, loss_v, sem, sem2):
    wid = lax.axis_index("s") * NC + lax.axis_index("c")
    base = wid * BPW
    la_copy = pltpu.async_copy(labels_hbm.at[pl.ds(base, BPW)], labels_v, sem)
    tc_copy = pltpu.async_copy(tgt_hbm, tgt_v, sem2)
    tc_copy.wait()
    t = tgt_v[...][0]
    toff = (((t >> 3) << 13) + ((base >> 7) << 10) + ((t & 7) << 7)
            + (base & 127))
    toff = pl.multiple_of(toff, 32)
    tg_copy = pltpu.async_copy(flat_hbm.at[pl.ds(toff, BPW)], tg_v, sem2)
    la_copy.wait()
    for j in range(BPW // L):
        lbl = labels_v[pl.ds(j * L, L)]
        b = base + j * L + lax.iota(jnp.int32, L)
        idx_v[pl.ds(j * L, L)] = (
            lax.shift_left(lax.shift_right_logical(lbl, 3), 13)
            + lax.shift_left(lax.shift_right_logical(b, 7), 10)
            + lax.shift_left(lbl & 7, 7)
            + (b & 127)
        )
    gt_copy = pltpu.async_copy(flat_hbm.at[idx_v], gt_v, sem)
    tg_copy.wait()
    gt_copy.wait()
    for j in range(BPW // L):
        loss_v[pl.ds(j * L, L)] = gt_v[pl.ds(j * L, L)] - tg_v[pl.ds(j * L, L)]
    pltpu.sync_copy(loss_v, out_hbm.at[pl.ds(base, BPW)])


def kernel(logits, labels, target):
    flat = logits.reshape(8, 128, V // 8, 8).transpose(2, 0, 3, 1).reshape(B * V)
    tgt_arr = jnp.full((L,), target, dtype=jnp.int32)
    mesh = plsc.VectorSubcoreMesh(core_axis_name="c", subcore_axis_name="s", num_cores=NC)
    k = pl.kernel(
        _body,
        out_type=jax.ShapeDtypeStruct((B,), jnp.float32),
        mesh=mesh,
        scratch_types=[
            pltpu.VMEM((BPW,), jnp.int32),
            pltpu.VMEM((L,), jnp.int32),
            pltpu.VMEM((BPW,), jnp.int32),
            pltpu.VMEM((BPW,), jnp.float32),
            pltpu.VMEM((BPW,), jnp.float32),
            pltpu.VMEM((BPW,), jnp.float32),
            pltpu.SemaphoreType.DMA,
            pltpu.SemaphoreType.DMA,
        ],
    )
    return k(flat, labels.astype(jnp.int32), tgt_arr)
